# speedup vs baseline: 1.5541x; 1.5541x over previous
import functools

import jax
import jax.numpy as jnp
from jax import lax
from jax.experimental import pallas as pl
from jax.experimental.pallas import tpu as pltpu
from jax.experimental.pallas import tpu_sc as plsc

NUM_POSITIONS = 8192
EMBED_DIM = 2048
B_TOTAL = 4 * 8192

NUM_CORES = 2
NUM_SUBCORES = 16
NW = NUM_CORES * NUM_SUBCORES
B_PER_W = B_TOTAL // NW
CHUNK = 16
NPAIR = B_PER_W // (2 * CHUNK)

_mesh = plsc.VectorSubcoreMesh(core_axis_name="c", subcore_axis_name="s")


@functools.partial(
    pl.kernel,
    mesh=_mesh,
    out_type=jax.ShapeDtypeStruct((B_TOTAL, EMBED_DIM), jnp.float32),
    scratch_types=[
        pltpu.VMEM((B_PER_W,), jnp.int32),
        pltpu.VMEM((CHUNK, EMBED_DIM), jnp.float32),
        pltpu.VMEM((CHUNK, EMBED_DIM), jnp.float32),
        pltpu.SemaphoreType.DMA,
        pltpu.SemaphoreType.DMA,
        pltpu.SemaphoreType.DMA,
        pltpu.SemaphoreType.DMA,
    ],
)
def _gather_kernel(idx_hbm, table_hbm, out_hbm, idx_v, rows0, rows1,
                   g0, g1, w0, w1):
    wid = lax.axis_index("s") * NUM_CORES + lax.axis_index("c")
    base = wid * B_PER_W
    pltpu.sync_copy(idx_hbm.at[pl.ds(base, B_PER_W)], idx_v)

    def gather_start(j, buf, sem):
        return pltpu.async_copy(
            table_hbm.at[idx_v.at[pl.ds(j * CHUNK, CHUNK)]],
            buf,
            sem,
        )

    def write_start(j, buf, sem):
        return pltpu.async_copy(
            buf,
            out_hbm.at[pl.ds(base + j * CHUNK, CHUNK)],
            sem,
        )

    def write_drain(buf, sem):
        pltpu.make_async_copy(
            buf,
            out_hbm.at[pl.ds(base, CHUNK)],
            sem,
        ).wait()

    gd0 = gather_start(0, rows0, g0)
    gd1 = gather_start(1, rows1, g1)
    gd0.wait()
    write_start(0, rows0, w0)
    gd1.wait()
    write_start(1, rows1, w1)

    def body(i, carry):
        j0 = 2 * i
        write_drain(rows0, w0)
        d0 = gather_start(j0, rows0, g0)
        write_drain(rows1, w1)
        d1 = gather_start(j0 + 1, rows1, g1)
        d0.wait()
        write_start(j0, rows0, w0)
        d1.wait()
        write_start(j0 + 1, rows1, w1)
        return carry

    lax.fori_loop(1, NPAIR, body, 0)
    write_drain(rows0, w0)
    write_drain(rows1, w1)


def kernel(position_ids, embedding_weight):
    flat = position_ids.reshape(-1).astype(jnp.int32)
    out = _gather_kernel(flat, embedding_weight)
    return out.reshape(position_ids.shape + (EMBED_DIM,))
